# baseline (speedup 1.0000x reference)
import functools

import jax
import jax.numpy as jnp
from jax import lax
from jax.experimental import pallas as pl
from jax.experimental.pallas import tpu as pltpu
from jax.experimental.pallas import tpu_sc as plsc

E = 8
TOP_K = 2
D_MODEL = 2048
D_FF = 2048
N = 8192
CAP = 2560
A = N * TOP_K
RR = 128
RC = A // RR
GROWS = E * CAP
DUMMY = GROWS
GPAD = GROWS + 8

BM = 320
BF = 2048
MB = CAP // BM
FB = D_FF // BF
BT = 512

CH = 16
DH = D_MODEL // 2


def _route_body(ei_ref, ew_ref, dest_ref, wadj_ref, counts_ref):
    ei = ei_ref[...]
    ew = ew_ref[...]
    r_i = lax.broadcasted_iota(jnp.int32, (RR, RC), 0)
    c_i = lax.broadcasted_iota(jnp.int32, (RR, RC), 1)
    su = (r_i < c_i).astype(jnp.float32)
    sl = (r_i > c_i).astype(jnp.float32)
    rank = jnp.zeros((RR, RC), jnp.float32)
    for e in range(E):
        m = (ei == e).astype(jnp.float32)
        within = jnp.dot(m, su, preferred_element_type=jnp.float32)
        rowsum = jnp.sum(m, axis=1, keepdims=True)
        rowpre = jnp.dot(sl, rowsum, preferred_element_type=jnp.float32)
        rank = jnp.where(ei == e, within + rowpre, rank)
        total = jnp.sum(m)
        counts_ref[e:e + 1, :] = (
            jnp.zeros((1, RC), jnp.float32) + total).astype(jnp.int32)
    ranki = rank.astype(jnp.int32)
    valid = ranki < CAP
    dest_ref[...] = jnp.where(valid, ei * CAP + ranki, DUMMY)
    wadj_ref[...] = jnp.where(valid, ew, 0.0)


def _route(ei2d, ew2d):
    return pl.pallas_call(
        _route_body,
        out_shape=[
            jax.ShapeDtypeStruct((RR, RC), jnp.int32),
            jax.ShapeDtypeStruct((RR, RC), jnp.float32),
            jax.ShapeDtypeStruct((E, RC), jnp.int32),
        ],
    )(ei2d, ew2d)


def _make_sc_mesh():
    return plsc.VectorSubcoreMesh(core_axis_name="c", subcore_axis_name="s")


@functools.lru_cache(maxsize=None)
def _scatter_x_call():
    mesh = _make_sc_mesh()
    ntiles = mesh.num_cores * mesh.num_subcores
    per = N // ntiles

    @functools.partial(
        pl.kernel,
        out_type=jax.ShapeDtypeStruct((GPAD, D_MODEL), jnp.float32),
        mesh=mesh,
        scratch_types=[
            pltpu.VMEM((CH, D_MODEL), jnp.float32),
            pltpu.VMEM((CH, D_MODEL), jnp.float32),
            pltpu.VMEM((CH,), jnp.int32),
            pltpu.VMEM((CH,), jnp.int32),
            pltpu.VMEM((CH,), jnp.int32),
            pltpu.VMEM((CH,), jnp.int32),
            pltpu.SemaphoreType.DMA,
            pltpu.SemaphoreType.DMA,
            pltpu.SemaphoreType.DMA,
            pltpu.SemaphoreType.DMA,
        ],
    )
    def scatter_x(x_hbm, d0_hbm, d1_hbm, g_hbm, rows_a, rows_b,
                  ia0, ia1, ib0, ib1, sa0, sa1, sb0, sb1):
        wid = lax.axis_index("s") * mesh.num_cores + lax.axis_index("c")
        base = wid * per

        def drain(rows, idx, sem):
            pltpu.make_async_copy(rows, g_hbm.at[idx], sem).wait()

        def body(j, carry):
            ta = base + (2 * j) * CH
            tb = ta + CH

            @pl.when(j > 0)
            def _():
                drain(rows_a, ia0, sa0)
                drain(rows_a, ia1, sa1)

            pltpu.sync_copy(x_hbm.at[pl.ds(ta, CH)], rows_a)
            pltpu.sync_copy(d0_hbm.at[pl.ds(ta, CH)], ia0)
            pltpu.sync_copy(d1_hbm.at[pl.ds(ta, CH)], ia1)
            pltpu.async_copy(rows_a, g_hbm.at[ia0], sa0)
            pltpu.async_copy(rows_a, g_hbm.at[ia1], sa1)

            @pl.when(j > 0)
            def _():
                drain(rows_b, ib0, sb0)
                drain(rows_b, ib1, sb1)

            pltpu.sync_copy(x_hbm.at[pl.ds(tb, CH)], rows_b)
            pltpu.sync_copy(d0_hbm.at[pl.ds(tb, CH)], ib0)
            pltpu.sync_copy(d1_hbm.at[pl.ds(tb, CH)], ib1)
            pltpu.async_copy(rows_b, g_hbm.at[ib0], sb0)
            pltpu.async_copy(rows_b, g_hbm.at[ib1], sb1)
            return carry

        lax.fori_loop(0, per // (2 * CH), body, 0)
        drain(rows_a, ia0, sa0)
        drain(rows_a, ia1, sa1)
        drain(rows_b, ib0, sb0)
        drain(rows_b, ib1, sb1)

    return scatter_x


def _mlp_body(g_ref, w1_ref, w2_ref, y_ref):
    g = g_ref[...].astype(jnp.bfloat16)
    h = jnp.dot(g, w1_ref[0], preferred_element_type=jnp.float32)
    h = jnp.maximum(h, 0.0).astype(jnp.bfloat16)
    y_ref[...] = jnp.dot(h, w2_ref[0], preferred_element_type=jnp.float32)


def _mlp(g, w1, w2):
    return pl.pallas_call(
        _mlp_body,
        grid=(E, MB),
        in_specs=[
            pl.BlockSpec((BM, D_MODEL), lambda e, m: (e * MB + m, 0)),
            pl.BlockSpec((1, D_MODEL, BF), lambda e, m: (e, 0, 0)),
            pl.BlockSpec((1, BF, D_MODEL), lambda e, m: (e, 0, 0)),
        ],
        out_specs=pl.BlockSpec((BM, D_MODEL), lambda e, m: (e * MB + m, 0)),
        out_shape=jax.ShapeDtypeStruct((GPAD, D_MODEL), jnp.float32),
        compiler_params=pltpu.CompilerParams(
            dimension_semantics=("parallel", "parallel")),
    )(g, w1, w2)


@functools.lru_cache(maxsize=None)
def _gather_y_call():
    mesh = _make_sc_mesh()
    ntiles = mesh.num_cores * mesh.num_subcores
    per = N // ntiles

    @functools.partial(
        pl.kernel,
        out_type=[
            jax.ShapeDtypeStruct((N, D_MODEL), jnp.float32),
            jax.ShapeDtypeStruct((N, D_MODEL), jnp.float32),
        ],
        mesh=mesh,
        scratch_types=[
            pltpu.VMEM((CH, D_MODEL), jnp.float32),
            pltpu.VMEM((CH, D_MODEL), jnp.float32),
            pltpu.VMEM((CH,), jnp.int32),
            pltpu.VMEM((CH,), jnp.int32),
            pltpu.SemaphoreType.DMA,
            pltpu.SemaphoreType.DMA,
            pltpu.SemaphoreType.DMA,
            pltpu.SemaphoreType.DMA,
        ],
    )
    def gather_y(y_hbm, d0_hbm, d1_hbm, z0_hbm, z1_hbm,
                 rows_a, rows_b, ia, ib, sga, sgb, swa, swb):
        wid = lax.axis_index("s") * mesh.num_cores + lax.axis_index("c")
        base = wid * per

        def body(j, carry):
            tb = base + j * CH

            @pl.when(j > 0)
            def _():
                pltpu.make_async_copy(
                    rows_a, z0_hbm.at[pl.ds(tb, CH)], swa).wait()

            pltpu.sync_copy(d0_hbm.at[pl.ds(tb, CH)], ia)
            pltpu.async_copy(y_hbm.at[ia], rows_a, sga)

            @pl.when(j > 0)
            def _():
                pltpu.make_async_copy(
                    rows_b, z1_hbm.at[pl.ds(tb, CH)], swb).wait()

            pltpu.sync_copy(d1_hbm.at[pl.ds(tb, CH)], ib)
            pltpu.async_copy(y_hbm.at[ib], rows_b, sgb)

            pltpu.make_async_copy(y_hbm.at[ia], rows_a, sga).wait()
            pltpu.async_copy(rows_a, z0_hbm.at[pl.ds(tb, CH)], swa)
            pltpu.make_async_copy(y_hbm.at[ib], rows_b, sgb).wait()
            pltpu.async_copy(rows_b, z1_hbm.at[pl.ds(tb, CH)], swb)
            return carry

        lax.fori_loop(0, per // CH, body, 0)
        pltpu.make_async_copy(rows_a, z0_hbm.at[pl.ds(0, CH)], swa).wait()
        pltpu.make_async_copy(rows_b, z1_hbm.at[pl.ds(0, CH)], swb).wait()

    return gather_y


def _combine_body(z0_ref, z1_ref, w0_ref, w1_ref, o_ref):
    w0 = w0_ref[...]
    w1 = w1_ref[...]
    o_ref[...] = (jnp.where(w0 > 0, z0_ref[...] * w0, 0.0)
                  + jnp.where(w1 > 0, z1_ref[...] * w1, 0.0))


def _combine(z0, z1, w0c, w1c):
    nb = N // BT
    return pl.pallas_call(
        _combine_body,
        grid=(nb,),
        in_specs=[
            pl.BlockSpec((BT, D_MODEL), lambda t: (t, 0)),
            pl.BlockSpec((BT, D_MODEL), lambda t: (t, 0)),
            pl.BlockSpec((BT, 1), lambda t: (t, 0)),
            pl.BlockSpec((BT, 1), lambda t: (t, 0)),
        ],
        out_specs=pl.BlockSpec((BT, D_MODEL), lambda t: (t, 0)),
        out_shape=jax.ShapeDtypeStruct((N, D_MODEL), jnp.float32),
        compiler_params=pltpu.CompilerParams(
            dimension_semantics=("parallel",)),
    )(z0, z1, w0c, w1c)


def kernel(x, expert_weights, expert_indices, w1, w2):
    ei2d = expert_indices.astype(jnp.int32).reshape(RR, RC)
    ew2d = expert_weights.astype(jnp.float32).reshape(RR, RC)
    dest2d, wadj2d, counts2d = _route(ei2d, ew2d)

    dest = dest2d.reshape(N, TOP_K)
    d0 = dest[:, 0]
    d1 = dest[:, 1]
    wadj = wadj2d.reshape(N, TOP_K)
    w0c = wadj[:, 0:1]
    w1c = wadj[:, 1:2]

    g = _scatter_x_call()(x, d0, d1)
    y = _mlp(g, w1.astype(jnp.bfloat16), w2.astype(jnp.bfloat16))
    z0, z1 = _gather_y_call()(y, d0, d1)
    out = _combine(z0, z1, w0c, w1c)
    counts = counts2d[:, 0]
    return out, counts


# speedup vs baseline: 1.0911x; 1.0911x over previous
import functools

import jax
import jax.numpy as jnp
from jax import lax
from jax.experimental import pallas as pl
from jax.experimental.pallas import tpu as pltpu
from jax.experimental.pallas import tpu_sc as plsc

E = 8
TOP_K = 2
D_MODEL = 2048
D_FF = 2048
N = 8192
CAP = 2560
A = N * TOP_K
RR = 128
RC = A // RR
GROWS = E * CAP
DUMMY = GROWS
GPAD = GROWS + 8

BM = 640
BF = 2048
MB = CAP // BM
FB = D_FF // BF
BT = 512

CH = 16
DH = D_MODEL // 2


def _route_body(ei_ref, ew_ref, dest_ref, wadj_ref, counts_ref):
    ei = ei_ref[...]
    ew = ew_ref[...]
    r_i = lax.broadcasted_iota(jnp.int32, (RR, RC), 0)
    c_i = lax.broadcasted_iota(jnp.int32, (RR, RC), 1)
    su = (r_i < c_i).astype(jnp.float32)
    sl = (r_i > c_i).astype(jnp.float32)
    rank = jnp.zeros((RR, RC), jnp.float32)
    for e in range(E):
        m = (ei == e).astype(jnp.float32)
        within = jnp.dot(m, su, preferred_element_type=jnp.float32)
        rowsum = jnp.sum(m, axis=1, keepdims=True)
        rowpre = jnp.dot(sl, rowsum, preferred_element_type=jnp.float32)
        rank = jnp.where(ei == e, within + rowpre, rank)
        total = jnp.sum(m)
        counts_ref[e:e + 1, :] = (
            jnp.zeros((1, RC), jnp.float32) + total).astype(jnp.int32)
    ranki = rank.astype(jnp.int32)
    valid = ranki < CAP
    dest_ref[...] = jnp.where(valid, ei * CAP + ranki, DUMMY)
    wadj_ref[...] = jnp.where(valid, ew, 0.0)


def _route(ei2d, ew2d):
    return pl.pallas_call(
        _route_body,
        out_shape=[
            jax.ShapeDtypeStruct((RR, RC), jnp.int32),
            jax.ShapeDtypeStruct((RR, RC), jnp.float32),
            jax.ShapeDtypeStruct((E, RC), jnp.int32),
        ],
    )(ei2d, ew2d)


def _make_sc_mesh():
    return plsc.VectorSubcoreMesh(core_axis_name="c", subcore_axis_name="s")


@functools.lru_cache(maxsize=None)
def _scatter_x_call():
    mesh = _make_sc_mesh()
    ntiles = mesh.num_cores * mesh.num_subcores
    per = N // ntiles

    @functools.partial(
        pl.kernel,
        out_type=jax.ShapeDtypeStruct((GPAD, DH), jnp.float32),
        mesh=mesh,
        scratch_types=[
            pltpu.VMEM((CH, DH), jnp.float32),
            pltpu.VMEM((CH, DH), jnp.float32),
            pltpu.VMEM((CH,), jnp.int32),
            pltpu.VMEM((CH,), jnp.int32),
            pltpu.VMEM((CH,), jnp.int32),
            pltpu.VMEM((CH,), jnp.int32),
            pltpu.SemaphoreType.DMA,
            pltpu.SemaphoreType.DMA,
            pltpu.SemaphoreType.DMA,
            pltpu.SemaphoreType.DMA,
        ],
    )
    def scatter_x(x_hbm, d0_hbm, d1_hbm, g_hbm, rows_a, rows_b,
                  ia0, ia1, ib0, ib1, sa0, sa1, sb0, sb1):
        wid = lax.axis_index("s") * mesh.num_cores + lax.axis_index("c")
        base = wid * per

        def drain(rows, idx, sem):
            pltpu.make_async_copy(rows, g_hbm.at[idx], sem).wait()

        def body(j, carry):
            ta = base + (2 * j) * CH
            tb = ta + CH

            @pl.when(j > 0)
            def _():
                drain(rows_a, ia0, sa0)
                drain(rows_a, ia1, sa1)

            pltpu.sync_copy(x_hbm.at[pl.ds(ta, CH)], rows_a)
            pltpu.sync_copy(d0_hbm.at[pl.ds(ta, CH)], ia0)
            pltpu.sync_copy(d1_hbm.at[pl.ds(ta, CH)], ia1)
            pltpu.async_copy(rows_a, g_hbm.at[ia0], sa0)
            pltpu.async_copy(rows_a, g_hbm.at[ia1], sa1)

            @pl.when(j > 0)
            def _():
                drain(rows_b, ib0, sb0)
                drain(rows_b, ib1, sb1)

            pltpu.sync_copy(x_hbm.at[pl.ds(tb, CH)], rows_b)
            pltpu.sync_copy(d0_hbm.at[pl.ds(tb, CH)], ib0)
            pltpu.sync_copy(d1_hbm.at[pl.ds(tb, CH)], ib1)
            pltpu.async_copy(rows_b, g_hbm.at[ib0], sb0)
            pltpu.async_copy(rows_b, g_hbm.at[ib1], sb1)
            return carry

        lax.fori_loop(0, per // (2 * CH), body, 0)
        drain(rows_a, ia0, sa0)
        drain(rows_a, ia1, sa1)
        drain(rows_b, ib0, sb0)
        drain(rows_b, ib1, sb1)

    return scatter_x


_HIMASK = 0xffff0000


def _unpack_pair(u):
    lo = lax.bitcast_convert_type(u << 16, jnp.float32)
    hi = lax.bitcast_convert_type(u & jnp.uint32(_HIMASK), jnp.float32)
    return lo, hi


def _pack_pair(lo_f32, hi_f32):
    ulo = lax.bitcast_convert_type(
        lo_f32.astype(jnp.bfloat16).astype(jnp.float32), jnp.uint32) >> 16
    uhi = lax.bitcast_convert_type(
        hi_f32.astype(jnp.bfloat16).astype(jnp.float32), jnp.uint32) & jnp.uint32(_HIMASK)
    return lax.bitcast_convert_type(ulo | uhi, jnp.float32)


def _mlp_body(g_ref, w1_ref, w2_ref, y_ref):
    u = lax.bitcast_convert_type(g_ref[...], jnp.uint32)
    glo, ghi = _unpack_pair(u)
    g = jnp.concatenate(
        [glo.astype(jnp.bfloat16), ghi.astype(jnp.bfloat16)], axis=1)
    h = jnp.dot(g, w1_ref[0], preferred_element_type=jnp.float32)
    h = jnp.maximum(h, 0.0).astype(jnp.bfloat16)
    y = jnp.dot(h, w2_ref[0], preferred_element_type=jnp.float32)
    y_ref[...] = _pack_pair(y[:, :DH], y[:, DH:])


def _mlp(g, w1, w2):
    return pl.pallas_call(
        _mlp_body,
        grid=(E, MB),
        in_specs=[
            pl.BlockSpec((BM, DH), lambda e, m: (e * MB + m, 0)),
            pl.BlockSpec((1, D_MODEL, BF), lambda e, m: (e, 0, 0)),
            pl.BlockSpec((1, BF, D_MODEL), lambda e, m: (e, 0, 0)),
        ],
        out_specs=pl.BlockSpec((BM, DH), lambda e, m: (e * MB + m, 0)),
        out_shape=jax.ShapeDtypeStruct((GPAD, DH), jnp.float32),
        compiler_params=pltpu.CompilerParams(
            dimension_semantics=("parallel", "parallel")),
    )(g, w1, w2)


@functools.lru_cache(maxsize=None)
def _gather_y_call():
    mesh = _make_sc_mesh()
    ntiles = mesh.num_cores * mesh.num_subcores
    per = N // ntiles

    @functools.partial(
        pl.kernel,
        out_type=[
            jax.ShapeDtypeStruct((N, DH), jnp.float32),
            jax.ShapeDtypeStruct((N, DH), jnp.float32),
        ],
        mesh=mesh,
        scratch_types=[
            pltpu.VMEM((CH, DH), jnp.float32),
            pltpu.VMEM((CH, DH), jnp.float32),
            pltpu.VMEM((CH,), jnp.int32),
            pltpu.VMEM((CH,), jnp.int32),
            pltpu.SemaphoreType.DMA,
            pltpu.SemaphoreType.DMA,
            pltpu.SemaphoreType.DMA,
            pltpu.SemaphoreType.DMA,
        ],
    )
    def gather_y(y_hbm, d0_hbm, d1_hbm, z0_hbm, z1_hbm,
                 rows_a, rows_b, ia, ib, sga, sgb, swa, swb):
        wid = lax.axis_index("s") * mesh.num_cores + lax.axis_index("c")
        base = wid * per

        def body(j, carry):
            tb = base + j * CH

            @pl.when(j > 0)
            def _():
                pltpu.make_async_copy(
                    rows_a, z0_hbm.at[pl.ds(tb, CH)], swa).wait()

            pltpu.sync_copy(d0_hbm.at[pl.ds(tb, CH)], ia)
            pltpu.async_copy(y_hbm.at[ia], rows_a, sga)

            @pl.when(j > 0)
            def _():
                pltpu.make_async_copy(
                    rows_b, z1_hbm.at[pl.ds(tb, CH)], swb).wait()

            pltpu.sync_copy(d1_hbm.at[pl.ds(tb, CH)], ib)
            pltpu.async_copy(y_hbm.at[ib], rows_b, sgb)

            pltpu.make_async_copy(y_hbm.at[ia], rows_a, sga).wait()
            pltpu.async_copy(rows_a, z0_hbm.at[pl.ds(tb, CH)], swa)
            pltpu.make_async_copy(y_hbm.at[ib], rows_b, sgb).wait()
            pltpu.async_copy(rows_b, z1_hbm.at[pl.ds(tb, CH)], swb)
            return carry

        lax.fori_loop(0, per // CH, body, 0)
        pltpu.make_async_copy(rows_a, z0_hbm.at[pl.ds(0, CH)], swa).wait()
        pltpu.make_async_copy(rows_b, z1_hbm.at[pl.ds(0, CH)], swb).wait()

    return gather_y


def _combine_body(z0_ref, z1_ref, w0_ref, w1_ref, o_ref):
    w0 = w0_ref[...]
    w1 = w1_ref[...]
    z0lo, z0hi = _unpack_pair(lax.bitcast_convert_type(z0_ref[...], jnp.uint32))
    z1lo, z1hi = _unpack_pair(lax.bitcast_convert_type(z1_ref[...], jnp.uint32))
    o_ref[:, :DH] = (jnp.where(w0 > 0, z0lo * w0, 0.0)
                     + jnp.where(w1 > 0, z1lo * w1, 0.0))
    o_ref[:, DH:] = (jnp.where(w0 > 0, z0hi * w0, 0.0)
                     + jnp.where(w1 > 0, z1hi * w1, 0.0))


def _combine(z0, z1, w0c, w1c):
    nb = N // BT
    return pl.pallas_call(
        _combine_body,
        grid=(nb,),
        in_specs=[
            pl.BlockSpec((BT, DH), lambda t: (t, 0)),
            pl.BlockSpec((BT, DH), lambda t: (t, 0)),
            pl.BlockSpec((BT, 1), lambda t: (t, 0)),
            pl.BlockSpec((BT, 1), lambda t: (t, 0)),
        ],
        out_specs=pl.BlockSpec((BT, D_MODEL), lambda t: (t, 0)),
        out_shape=jax.ShapeDtypeStruct((N, D_MODEL), jnp.float32),
        compiler_params=pltpu.CompilerParams(
            dimension_semantics=("parallel",)),
    )(z0, z1, w0c, w1c)


def kernel(x, expert_weights, expert_indices, w1, w2):
    ei2d = expert_indices.astype(jnp.int32).reshape(RR, RC)
    ew2d = expert_weights.astype(jnp.float32).reshape(RR, RC)
    dest2d, wadj2d, counts2d = _route(ei2d, ew2d)

    dest = dest2d.reshape(N, TOP_K)
    d0 = dest[:, 0]
    d1 = dest[:, 1]
    wadj = wadj2d.reshape(N, TOP_K)
    w0c = wadj[:, 0:1]
    w1c = wadj[:, 1:2]

    xb = x.astype(jnp.bfloat16)
    xlo = lax.bitcast_convert_type(xb[:, :DH], jnp.uint16).astype(jnp.uint32)
    xhi = lax.bitcast_convert_type(xb[:, DH:], jnp.uint16).astype(jnp.uint32)
    xp = lax.bitcast_convert_type(xlo | (xhi << 16), jnp.float32)
    g = _scatter_x_call()(xp, d0, d1)
    y = _mlp(g, w1.astype(jnp.bfloat16), w2.astype(jnp.bfloat16))
    z0, z1 = _gather_y_call()(y, d0, d1)
    out = _combine(z0, z1, w0c, w1c)
    counts = counts2d[:, 0]
    return out, counts


# speedup vs baseline: 1.0948x; 1.0034x over previous
import functools

import jax
import jax.numpy as jnp
from jax import lax
from jax.experimental import pallas as pl
from jax.experimental.pallas import tpu as pltpu
from jax.experimental.pallas import tpu_sc as plsc

E = 8
TOP_K = 2
D_MODEL = 2048
D_FF = 2048
N = 8192
CAP = 2560
A = N * TOP_K
RR = 128
RC = A // RR
GROWS = E * CAP
DUMMY = GROWS
GPAD = GROWS + 8

BM = 640
BF = 2048
MB = CAP // BM
FB = D_FF // BF
BT = 512

CH = 32
DH = D_MODEL // 2


def _route_body(ei_ref, ew_ref, dest_ref, wadj_ref, counts_ref):
    ei = ei_ref[...]
    ew = ew_ref[...]
    r_i = lax.broadcasted_iota(jnp.int32, (RR, RC), 0)
    c_i = lax.broadcasted_iota(jnp.int32, (RR, RC), 1)
    su = (r_i < c_i).astype(jnp.float32)
    sl = (r_i > c_i).astype(jnp.float32)
    rank = jnp.zeros((RR, RC), jnp.float32)
    for e in range(E):
        m = (ei == e).astype(jnp.float32)
        within = jnp.dot(m, su, preferred_element_type=jnp.float32)
        rowsum = jnp.sum(m, axis=1, keepdims=True)
        rowpre = jnp.dot(sl, rowsum, preferred_element_type=jnp.float32)
        rank = jnp.where(ei == e, within + rowpre, rank)
        total = jnp.sum(m)
        counts_ref[e:e + 1, :] = (
            jnp.zeros((1, RC), jnp.float32) + total).astype(jnp.int32)
    ranki = rank.astype(jnp.int32)
    valid = ranki < CAP
    dest_ref[...] = jnp.where(valid, ei * CAP + ranki, DUMMY)
    wadj_ref[...] = jnp.where(valid, ew, 0.0)


def _route(ei2d, ew2d):
    return pl.pallas_call(
        _route_body,
        out_shape=[
            jax.ShapeDtypeStruct((RR, RC), jnp.int32),
            jax.ShapeDtypeStruct((RR, RC), jnp.float32),
            jax.ShapeDtypeStruct((E, RC), jnp.int32),
        ],
    )(ei2d, ew2d)


def _make_sc_mesh():
    return plsc.VectorSubcoreMesh(core_axis_name="c", subcore_axis_name="s")


@functools.lru_cache(maxsize=None)
def _scatter_x_call():
    mesh = _make_sc_mesh()
    ntiles = mesh.num_cores * mesh.num_subcores
    per = N // ntiles

    @functools.partial(
        pl.kernel,
        out_type=jax.ShapeDtypeStruct((GPAD, DH), jnp.float32),
        mesh=mesh,
        scratch_types=[
            pltpu.VMEM((CH, DH), jnp.float32),
            pltpu.VMEM((CH, DH), jnp.float32),
            pltpu.VMEM((CH,), jnp.int32),
            pltpu.VMEM((CH,), jnp.int32),
            pltpu.VMEM((CH,), jnp.int32),
            pltpu.VMEM((CH,), jnp.int32),
            pltpu.SemaphoreType.DMA,
            pltpu.SemaphoreType.DMA,
            pltpu.SemaphoreType.DMA,
            pltpu.SemaphoreType.DMA,
        ],
    )
    def scatter_x(x_hbm, d0_hbm, d1_hbm, g_hbm, rows_a, rows_b,
                  ia0, ia1, ib0, ib1, sa0, sa1, sb0, sb1):
        wid = lax.axis_index("s") * mesh.num_cores + lax.axis_index("c")
        base = wid * per

        def drain(rows, idx, sem):
            pltpu.make_async_copy(rows, g_hbm.at[idx], sem).wait()

        def body(j, carry):
            ta = base + (2 * j) * CH
            tb = ta + CH

            @pl.when(j > 0)
            def _():
                drain(rows_a, ia0, sa0)
                drain(rows_a, ia1, sa1)

            pltpu.sync_copy(x_hbm.at[pl.ds(ta, CH)], rows_a)
            pltpu.sync_copy(d0_hbm.at[pl.ds(ta, CH)], ia0)
            pltpu.sync_copy(d1_hbm.at[pl.ds(ta, CH)], ia1)
            pltpu.async_copy(rows_a, g_hbm.at[ia0], sa0)
            pltpu.async_copy(rows_a, g_hbm.at[ia1], sa1)

            @pl.when(j > 0)
            def _():
                drain(rows_b, ib0, sb0)
                drain(rows_b, ib1, sb1)

            pltpu.sync_copy(x_hbm.at[pl.ds(tb, CH)], rows_b)
            pltpu.sync_copy(d0_hbm.at[pl.ds(tb, CH)], ib0)
            pltpu.sync_copy(d1_hbm.at[pl.ds(tb, CH)], ib1)
            pltpu.async_copy(rows_b, g_hbm.at[ib0], sb0)
            pltpu.async_copy(rows_b, g_hbm.at[ib1], sb1)
            return carry

        lax.fori_loop(0, per // (2 * CH), body, 0)
        drain(rows_a, ia0, sa0)
        drain(rows_a, ia1, sa1)
        drain(rows_b, ib0, sb0)
        drain(rows_b, ib1, sb1)

    return scatter_x


_HIMASK = 0xffff0000


def _unpack_pair(u):
    lo = lax.bitcast_convert_type(u << 16, jnp.float32)
    hi = lax.bitcast_convert_type(u & jnp.uint32(_HIMASK), jnp.float32)
    return lo, hi


def _pack_pair(lo_f32, hi_f32):
    ulo = lax.bitcast_convert_type(
        lo_f32.astype(jnp.bfloat16).astype(jnp.float32), jnp.uint32) >> 16
    uhi = lax.bitcast_convert_type(
        hi_f32.astype(jnp.bfloat16).astype(jnp.float32), jnp.uint32) & jnp.uint32(_HIMASK)
    return lax.bitcast_convert_type(ulo | uhi, jnp.float32)


def _mlp_body(g_ref, w1_ref, w2_ref, y_ref):
    u = lax.bitcast_convert_type(g_ref[...], jnp.uint32)
    glo, ghi = _unpack_pair(u)
    g = jnp.concatenate(
        [glo.astype(jnp.bfloat16), ghi.astype(jnp.bfloat16)], axis=1)
    h = jnp.dot(g, w1_ref[0], preferred_element_type=jnp.float32)
    h = jnp.maximum(h, 0.0).astype(jnp.bfloat16)
    y = jnp.dot(h, w2_ref[0], preferred_element_type=jnp.float32)
    y_ref[...] = _pack_pair(y[:, :DH], y[:, DH:])


def _mlp(g, w1, w2):
    return pl.pallas_call(
        _mlp_body,
        grid=(E, MB),
        in_specs=[
            pl.BlockSpec((BM, DH), lambda e, m: (e * MB + m, 0)),
            pl.BlockSpec((1, D_MODEL, BF), lambda e, m: (e, 0, 0)),
            pl.BlockSpec((1, BF, D_MODEL), lambda e, m: (e, 0, 0)),
        ],
        out_specs=pl.BlockSpec((BM, DH), lambda e, m: (e * MB + m, 0)),
        out_shape=jax.ShapeDtypeStruct((GPAD, DH), jnp.float32),
        compiler_params=pltpu.CompilerParams(
            dimension_semantics=("parallel", "parallel")),
    )(g, w1, w2)


@functools.lru_cache(maxsize=None)
def _gather_y_call():
    mesh = _make_sc_mesh()
    ntiles = mesh.num_cores * mesh.num_subcores
    per = N // ntiles

    @functools.partial(
        pl.kernel,
        out_type=[
            jax.ShapeDtypeStruct((N, DH), jnp.float32),
            jax.ShapeDtypeStruct((N, DH), jnp.float32),
        ],
        mesh=mesh,
        scratch_types=[
            pltpu.VMEM((CH, DH), jnp.float32),
            pltpu.VMEM((CH, DH), jnp.float32),
            pltpu.VMEM((CH,), jnp.int32),
            pltpu.VMEM((CH,), jnp.int32),
            pltpu.SemaphoreType.DMA,
            pltpu.SemaphoreType.DMA,
            pltpu.SemaphoreType.DMA,
            pltpu.SemaphoreType.DMA,
        ],
    )
    def gather_y(y_hbm, d0_hbm, d1_hbm, z0_hbm, z1_hbm,
                 rows_a, rows_b, ia, ib, sga, sgb, swa, swb):
        wid = lax.axis_index("s") * mesh.num_cores + lax.axis_index("c")
        base = wid * per

        def body(j, carry):
            tb = base + j * CH

            @pl.when(j > 0)
            def _():
                pltpu.make_async_copy(
                    rows_a, z0_hbm.at[pl.ds(tb, CH)], swa).wait()

            pltpu.sync_copy(d0_hbm.at[pl.ds(tb, CH)], ia)
            pltpu.async_copy(y_hbm.at[ia], rows_a, sga)

            @pl.when(j > 0)
            def _():
                pltpu.make_async_copy(
                    rows_b, z1_hbm.at[pl.ds(tb, CH)], swb).wait()

            pltpu.sync_copy(d1_hbm.at[pl.ds(tb, CH)], ib)
            pltpu.async_copy(y_hbm.at[ib], rows_b, sgb)

            pltpu.make_async_copy(y_hbm.at[ia], rows_a, sga).wait()
            pltpu.async_copy(rows_a, z0_hbm.at[pl.ds(tb, CH)], swa)
            pltpu.make_async_copy(y_hbm.at[ib], rows_b, sgb).wait()
            pltpu.async_copy(rows_b, z1_hbm.at[pl.ds(tb, CH)], swb)
            return carry

        lax.fori_loop(0, per // CH, body, 0)
        pltpu.make_async_copy(rows_a, z0_hbm.at[pl.ds(0, CH)], swa).wait()
        pltpu.make_async_copy(rows_b, z1_hbm.at[pl.ds(0, CH)], swb).wait()

    return gather_y


def _combine_body(z0_ref, z1_ref, w0_ref, w1_ref, o_ref):
    w0 = w0_ref[...]
    w1 = w1_ref[...]
    z0lo, z0hi = _unpack_pair(lax.bitcast_convert_type(z0_ref[...], jnp.uint32))
    z1lo, z1hi = _unpack_pair(lax.bitcast_convert_type(z1_ref[...], jnp.uint32))
    o_ref[:, :DH] = (jnp.where(w0 > 0, z0lo * w0, 0.0)
                     + jnp.where(w1 > 0, z1lo * w1, 0.0))
    o_ref[:, DH:] = (jnp.where(w0 > 0, z0hi * w0, 0.0)
                     + jnp.where(w1 > 0, z1hi * w1, 0.0))


def _combine(z0, z1, w0c, w1c):
    nb = N // BT
    return pl.pallas_call(
        _combine_body,
        grid=(nb,),
        in_specs=[
            pl.BlockSpec((BT, DH), lambda t: (t, 0)),
            pl.BlockSpec((BT, DH), lambda t: (t, 0)),
            pl.BlockSpec((BT, 1), lambda t: (t, 0)),
            pl.BlockSpec((BT, 1), lambda t: (t, 0)),
        ],
        out_specs=pl.BlockSpec((BT, D_MODEL), lambda t: (t, 0)),
        out_shape=jax.ShapeDtypeStruct((N, D_MODEL), jnp.float32),
        compiler_params=pltpu.CompilerParams(
            dimension_semantics=("parallel",)),
    )(z0, z1, w0c, w1c)


def kernel(x, expert_weights, expert_indices, w1, w2):
    ei2d = expert_indices.astype(jnp.int32).reshape(RR, RC)
    ew2d = expert_weights.astype(jnp.float32).reshape(RR, RC)
    dest2d, wadj2d, counts2d = _route(ei2d, ew2d)

    dest = dest2d.reshape(N, TOP_K)
    d0 = dest[:, 0]
    d1 = dest[:, 1]
    wadj = wadj2d.reshape(N, TOP_K)
    w0c = wadj[:, 0:1]
    w1c = wadj[:, 1:2]

    xb = x.astype(jnp.bfloat16)
    xlo = lax.bitcast_convert_type(xb[:, :DH], jnp.uint16).astype(jnp.uint32)
    xhi = lax.bitcast_convert_type(xb[:, DH:], jnp.uint16).astype(jnp.uint32)
    xp = lax.bitcast_convert_type(xlo | (xhi << 16), jnp.float32)
    g = _scatter_x_call()(xp, d0, d1)
    y = _mlp(g, w1.astype(jnp.bfloat16), w2.astype(jnp.bfloat16))
    z0, z1 = _gather_y_call()(y, d0, d1)
    out = _combine(z0, z1, w0c, w1c)
    counts = counts2d[:, 0]
    return out, counts


# speedup vs baseline: 1.1658x; 1.0648x over previous
import functools

import jax
import jax.numpy as jnp
from jax import lax
from jax.experimental import pallas as pl
from jax.experimental.pallas import tpu as pltpu
from jax.experimental.pallas import tpu_sc as plsc

E = 8
TOP_K = 2
D_MODEL = 2048
D_FF = 2048
N = 8192
CAP = 2560
A = N * TOP_K
RR = 128
RC = A // RR
GROWS = E * CAP
DUMMY = GROWS
GPAD = GROWS + 8

BM = 640
BF = 1024
MB = CAP // BM
FB = D_FF // BF
BT = 512

CH = 32
DH = D_MODEL // 2


def _route_body(ei_ref, ew_ref, dest_ref, wadj_ref, counts_ref):
    ei = ei_ref[...]
    ew = ew_ref[...]
    r_i = lax.broadcasted_iota(jnp.int32, (RR, RC), 0)
    c_i = lax.broadcasted_iota(jnp.int32, (RR, RC), 1)
    su = (r_i < c_i).astype(jnp.float32)
    sl = (r_i > c_i).astype(jnp.float32)
    rank = jnp.zeros((RR, RC), jnp.float32)
    for e in range(E):
        m = (ei == e).astype(jnp.float32)
        within = jnp.dot(m, su, preferred_element_type=jnp.float32)
        rowsum = jnp.sum(m, axis=1, keepdims=True)
        rowpre = jnp.dot(sl, rowsum, preferred_element_type=jnp.float32)
        rank = jnp.where(ei == e, within + rowpre, rank)
        total = jnp.sum(m)
        counts_ref[e:e + 1, :] = (
            jnp.zeros((1, RC), jnp.float32) + total).astype(jnp.int32)
    ranki = rank.astype(jnp.int32)
    valid = ranki < CAP
    dest_ref[...] = jnp.where(valid, ei * CAP + ranki, DUMMY)
    wadj_ref[...] = jnp.where(valid, ew, 0.0)


def _route(ei2d, ew2d):
    return pl.pallas_call(
        _route_body,
        out_shape=[
            jax.ShapeDtypeStruct((RR, RC), jnp.int32),
            jax.ShapeDtypeStruct((RR, RC), jnp.float32),
            jax.ShapeDtypeStruct((E, RC), jnp.int32),
        ],
    )(ei2d, ew2d)


def _make_sc_mesh():
    return plsc.VectorSubcoreMesh(core_axis_name="c", subcore_axis_name="s")


@functools.lru_cache(maxsize=None)
def _scatter_x_call():
    mesh = _make_sc_mesh()
    ntiles = mesh.num_cores * mesh.num_subcores
    per = N // ntiles

    @functools.partial(
        pl.kernel,
        out_type=jax.ShapeDtypeStruct((GPAD, DH), jnp.float32),
        mesh=mesh,
        scratch_types=[
            pltpu.VMEM((CH, DH), jnp.float32),
            pltpu.VMEM((CH, DH), jnp.float32),
            pltpu.VMEM((CH,), jnp.int32),
            pltpu.VMEM((CH,), jnp.int32),
            pltpu.VMEM((CH,), jnp.int32),
            pltpu.VMEM((CH,), jnp.int32),
            pltpu.SemaphoreType.DMA,
            pltpu.SemaphoreType.DMA,
            pltpu.SemaphoreType.DMA,
            pltpu.SemaphoreType.DMA,
        ],
    )
    def scatter_x(x_hbm, d0_hbm, d1_hbm, g_hbm, rows_a, rows_b,
                  ia0, ia1, ib0, ib1, sa0, sa1, sb0, sb1):
        wid = lax.axis_index("s") * mesh.num_cores + lax.axis_index("c")
        base = wid * per

        def drain(rows, idx, sem):
            pltpu.make_async_copy(rows, g_hbm.at[idx], sem).wait()

        def body(j, carry):
            ta = base + (2 * j) * CH
            tb = ta + CH

            @pl.when(j > 0)
            def _():
                drain(rows_a, ia0, sa0)
                drain(rows_a, ia1, sa1)

            pltpu.sync_copy(x_hbm.at[pl.ds(ta, CH)], rows_a)
            pltpu.sync_copy(d0_hbm.at[pl.ds(ta, CH)], ia0)
            pltpu.sync_copy(d1_hbm.at[pl.ds(ta, CH)], ia1)
            pltpu.async_copy(rows_a, g_hbm.at[ia0], sa0)
            pltpu.async_copy(rows_a, g_hbm.at[ia1], sa1)

            @pl.when(j > 0)
            def _():
                drain(rows_b, ib0, sb0)
                drain(rows_b, ib1, sb1)

            pltpu.sync_copy(x_hbm.at[pl.ds(tb, CH)], rows_b)
            pltpu.sync_copy(d0_hbm.at[pl.ds(tb, CH)], ib0)
            pltpu.sync_copy(d1_hbm.at[pl.ds(tb, CH)], ib1)
            pltpu.async_copy(rows_b, g_hbm.at[ib0], sb0)
            pltpu.async_copy(rows_b, g_hbm.at[ib1], sb1)
            return carry

        lax.fori_loop(0, per // (2 * CH), body, 0)
        drain(rows_a, ia0, sa0)
        drain(rows_a, ia1, sa1)
        drain(rows_b, ib0, sb0)
        drain(rows_b, ib1, sb1)

    return scatter_x


_HIMASK = 0xffff0000


def _unpack_pair(u):
    lo = lax.bitcast_convert_type(u << 16, jnp.float32)
    hi = lax.bitcast_convert_type(u & jnp.uint32(_HIMASK), jnp.float32)
    return lo, hi


def _pack_pair(lo_f32, hi_f32):
    ulo = lax.bitcast_convert_type(
        lo_f32.astype(jnp.bfloat16).astype(jnp.float32), jnp.uint32) >> 16
    uhi = lax.bitcast_convert_type(
        hi_f32.astype(jnp.bfloat16).astype(jnp.float32), jnp.uint32) & jnp.uint32(_HIMASK)
    return lax.bitcast_convert_type(ulo | uhi, jnp.float32)


def _mlp_body(g_ref, w1_ref, w2_ref, y_ref, acc_ref):
    f = pl.program_id(2)
    u = lax.bitcast_convert_type(g_ref[...], jnp.uint32)
    glo, ghi = _unpack_pair(u)
    g = jnp.concatenate(
        [glo.astype(jnp.bfloat16), ghi.astype(jnp.bfloat16)], axis=1)
    h = jnp.dot(g, w1_ref[0].astype(jnp.bfloat16),
                preferred_element_type=jnp.float32)
    h = jnp.maximum(h, 0.0).astype(jnp.bfloat16)
    y = jnp.dot(h, w2_ref[0].astype(jnp.bfloat16),
                preferred_element_type=jnp.float32)

    @pl.when(f == 0)
    def _():
        acc_ref[...] = y

    @pl.when(f != 0)
    def _():
        acc_ref[...] += y

    @pl.when(f == FB - 1)
    def _():
        ya = acc_ref[...]
        y_ref[...] = _pack_pair(ya[:, :DH], ya[:, DH:])


def _mlp(g, w1, w2):
    return pl.pallas_call(
        _mlp_body,
        grid=(E, MB, FB),
        in_specs=[
            pl.BlockSpec((BM, DH), lambda e, m, f: (e * MB + m, 0)),
            pl.BlockSpec((1, D_MODEL, BF), lambda e, m, f: (e, 0, f)),
            pl.BlockSpec((1, BF, D_MODEL), lambda e, m, f: (e, f, 0)),
        ],
        out_specs=pl.BlockSpec((BM, DH), lambda e, m, f: (e * MB + m, 0)),
        out_shape=jax.ShapeDtypeStruct((GPAD, DH), jnp.float32),
        scratch_shapes=[pltpu.VMEM((BM, D_MODEL), jnp.float32)],
        compiler_params=pltpu.CompilerParams(
            dimension_semantics=("parallel", "parallel", "arbitrary")),
    )(g, w1, w2)


@functools.lru_cache(maxsize=None)
def _gather_y_call():
    mesh = _make_sc_mesh()
    ntiles = mesh.num_cores * mesh.num_subcores
    per = N // ntiles

    @functools.partial(
        pl.kernel,
        out_type=[
            jax.ShapeDtypeStruct((N, DH), jnp.float32),
            jax.ShapeDtypeStruct((N, DH), jnp.float32),
        ],
        mesh=mesh,
        scratch_types=[
            pltpu.VMEM((CH, DH), jnp.float32),
            pltpu.VMEM((CH, DH), jnp.float32),
            pltpu.VMEM((CH,), jnp.int32),
            pltpu.VMEM((CH,), jnp.int32),
            pltpu.SemaphoreType.DMA,
            pltpu.SemaphoreType.DMA,
            pltpu.SemaphoreType.DMA,
            pltpu.SemaphoreType.DMA,
        ],
    )
    def gather_y(y_hbm, d0_hbm, d1_hbm, z0_hbm, z1_hbm,
                 rows_a, rows_b, ia, ib, sga, sgb, swa, swb):
        wid = lax.axis_index("s") * mesh.num_cores + lax.axis_index("c")
        base = wid * per

        def body(j, carry):
            tb = base + j * CH

            @pl.when(j > 0)
            def _():
                pltpu.make_async_copy(
                    rows_a, z0_hbm.at[pl.ds(tb, CH)], swa).wait()

            pltpu.sync_copy(d0_hbm.at[pl.ds(tb, CH)], ia)
            pltpu.async_copy(y_hbm.at[ia], rows_a, sga)

            @pl.when(j > 0)
            def _():
                pltpu.make_async_copy(
                    rows_b, z1_hbm.at[pl.ds(tb, CH)], swb).wait()

            pltpu.sync_copy(d1_hbm.at[pl.ds(tb, CH)], ib)
            pltpu.async_copy(y_hbm.at[ib], rows_b, sgb)

            pltpu.make_async_copy(y_hbm.at[ia], rows_a, sga).wait()
            pltpu.async_copy(rows_a, z0_hbm.at[pl.ds(tb, CH)], swa)
            pltpu.make_async_copy(y_hbm.at[ib], rows_b, sgb).wait()
            pltpu.async_copy(rows_b, z1_hbm.at[pl.ds(tb, CH)], swb)
            return carry

        lax.fori_loop(0, per // CH, body, 0)
        pltpu.make_async_copy(rows_a, z0_hbm.at[pl.ds(0, CH)], swa).wait()
        pltpu.make_async_copy(rows_b, z1_hbm.at[pl.ds(0, CH)], swb).wait()

    return gather_y


def _combine_body(z0_ref, z1_ref, w0_ref, w1_ref, o_ref):
    w0 = w0_ref[...]
    w1 = w1_ref[...]
    z0lo, z0hi = _unpack_pair(lax.bitcast_convert_type(z0_ref[...], jnp.uint32))
    z1lo, z1hi = _unpack_pair(lax.bitcast_convert_type(z1_ref[...], jnp.uint32))
    o_ref[:, :DH] = (jnp.where(w0 > 0, z0lo * w0, 0.0)
                     + jnp.where(w1 > 0, z1lo * w1, 0.0))
    o_ref[:, DH:] = (jnp.where(w0 > 0, z0hi * w0, 0.0)
                     + jnp.where(w1 > 0, z1hi * w1, 0.0))


def _combine(z0, z1, w0c, w1c):
    nb = N // BT
    return pl.pallas_call(
        _combine_body,
        grid=(nb,),
        in_specs=[
            pl.BlockSpec((BT, DH), lambda t: (t, 0)),
            pl.BlockSpec((BT, DH), lambda t: (t, 0)),
            pl.BlockSpec((BT, 1), lambda t: (t, 0)),
            pl.BlockSpec((BT, 1), lambda t: (t, 0)),
        ],
        out_specs=pl.BlockSpec((BT, D_MODEL), lambda t: (t, 0)),
        out_shape=jax.ShapeDtypeStruct((N, D_MODEL), jnp.float32),
        compiler_params=pltpu.CompilerParams(
            dimension_semantics=("parallel",)),
    )(z0, z1, w0c, w1c)


def kernel(x, expert_weights, expert_indices, w1, w2):
    ei2d = expert_indices.astype(jnp.int32).reshape(RR, RC)
    ew2d = expert_weights.astype(jnp.float32).reshape(RR, RC)
    dest2d, wadj2d, counts2d = _route(ei2d, ew2d)

    dest = dest2d.reshape(N, TOP_K)
    d0 = dest[:, 0]
    d1 = dest[:, 1]
    wadj = wadj2d.reshape(N, TOP_K)
    w0c = wadj[:, 0:1]
    w1c = wadj[:, 1:2]

    xb = x.astype(jnp.bfloat16)
    xlo = lax.bitcast_convert_type(xb[:, :DH], jnp.uint16).astype(jnp.uint32)
    xhi = lax.bitcast_convert_type(xb[:, DH:], jnp.uint16).astype(jnp.uint32)
    xp = lax.bitcast_convert_type(xlo | (xhi << 16), jnp.float32)
    g = _scatter_x_call()(xp, d0, d1)
    y = _mlp(g, w1, w2)
    z0, z1 = _gather_y_call()(y, d0, d1)
    out = _combine(z0, z1, w0c, w1c)
    counts = counts2d[:, 0]
    return out, counts
